# baseline (speedup 1.0000x reference)
import functools

import numpy as np
import jax
import jax.numpy as jnp
from jax import lax
from jax.experimental import pallas as pl
from jax.experimental.pallas import tpu as pltpu
from jax.experimental.pallas import tpu_sc as plsc

NC = 2
NS = 16
NW = NC * NS
CH = 2000
GP = CH // 16

_TWO_PI = np.float32(2.0 * np.pi)


def _lk(v):
    return jnp.maximum(v, np.float32(0.01) * v)


def _edge_body(efl_hbm, f_hbm, a_hbm, b_hbm, w_hbm, out_hbm,
               *scr, NR, EW, n_chunks, ZR, EOFF):
    sidx = scr[0:2]
    didx = scr[2:4]
    dsc = scr[4:6]
    gb = (scr[6:12], scr[12:18])
    mb = (scr[18:23], scr[23:28])
    onesb, wbuf = scr[28:30]
    fsh, ash, bsh = scr[30:33]
    acc = scr[33:39]
    isem = scr[39:41]
    gsem = (scr[41:47], scr[47:53])
    ssem = (scr[53:59], scr[59:65])
    stg = gb[0][0]

    segs = []
    off = 0
    while off < ZR:
        segs.append((off, min(CH, ZR - off)))
        off += CH

    c = lax.axis_index("c")
    s = lax.axis_index("s")
    wid = c * NS + s

    rs = s * ZR
    for hsrc, shdst in ((f_hbm, fsh), (a_hbm, ash), (b_hbm, bsh)):
        for off, ln in segs:
            pltpu.sync_copy(hsrc.at[pl.ds(rs + off, ln)], stg.at[pl.ds(0, ln)])
            pltpu.sync_copy(stg.at[pl.ds(0, ln)], shdst.at[pl.ds(rs + off, ln)])

    def zloop(t, carry):
        stg[pl.ds(t * 16, 16)] = jnp.zeros((16,), jnp.float32)
        return carry
    lax.fori_loop(0, CH // 16, zloop, 0)
    for ac in acc:
        for off, ln in segs:
            pltpu.sync_copy(stg.at[pl.ds(0, ln)], ac.at[pl.ds(rs + off, ln)])

    pltpu.sync_copy(w_hbm, wbuf)
    for g in range(GP):
        onesb[pl.ds(g * 16, 16)] = jnp.ones((16,), jnp.float32)

    plsc.subcore_barrier()

    ebase = wid * EW

    def issue_idx(cid, p):
        base = ebase + cid * CH
        pltpu.async_copy(efl_hbm.at[pl.ds(base, CH)], sidx[p], isem[p])
        pltpu.async_copy(efl_hbm.at[pl.ds(EOFF + base, CH)], didx[p], isem[p])

    def wait_idx(p):
        pltpu.make_async_copy(efl_hbm.at[pl.ds(0, CH)], sidx[p], isem[p]).wait()
        pltpu.make_async_copy(efl_hbm.at[pl.ds(0, CH)], didx[p], isem[p]).wait()

    def gather_list(p):
        return ((fsh.at[didx[p]], gb[p][0]), (ash.at[didx[p]], gb[p][1]),
                (bsh.at[didx[p]], gb[p][2]), (fsh.at[sidx[p]], gb[p][3]),
                (ash.at[sidx[p]], gb[p][4]), (bsh.at[sidx[p]], gb[p][5]))

    def issue_gathers(p):
        for k, (s_, d_) in enumerate(gather_list(p)):
            pltpu.async_copy(s_, d_, gsem[p][k])

    def wait_gathers(p):
        for k, (s_, d_) in enumerate(gather_list(p)):
            pltpu.make_async_copy(s_, d_, gsem[p][k]).wait()

    def scatter_list(p):
        return ((mb[p][0], acc[0]), (mb[p][1], acc[1]), (mb[p][2], acc[2]),
                (mb[p][3], acc[3]), (mb[p][4], acc[4]), (onesb, acc[5]))

    def issue_scatters(p):
        for k, (s_, d_) in enumerate(scatter_list(p)):
            pltpu.async_copy(s_, d_.at[dsc[p]], ssem[p][k], add=True)

    def wait_scatters(p):
        for k, (s_, d_) in enumerate(scatter_list(p)):
            pltpu.make_async_copy(s_, d_.at[dsc[p]], ssem[p][k]).wait()

    def compute(p):
        gfi, gai, gbi, gfj, gaj, gbj = gb[p]

        w1 = [wbuf[k, :] for k in range(25)]

        def layer1(g, carry):
            sl = pl.ds(g * 16, 16)
            dsc[p][sl] = didx[p][sl]
            fi = gfi[sl]
            ai = gai[sl]
            bi = gbi[sl]
            fj = gfj[sl]
            aj = gaj[sl]
            bj = gbj[sl]
            inv = np.float32(1.0) / (aj * aj + bj * bj)
            re = (ai * aj + bi * bj) * inv
            im = (bi * aj - ai * bj) * inv
            for k in range(5):
                mb[p][k][sl] = _lk(w1[20 + k] + fi * w1[k] + fj * w1[5 + k]
                                   + re * w1[10 + k] + im * w1[15 + k])
            return carry

        lax.fori_loop(0, GP, layer1, 0)

        w2 = [wbuf[25 + k, :] for k in range(30)]

        def layer2(g, carry):
            sl = pl.ds(g * 16, 16)
            h = [mb[p][k][sl] for k in range(5)]
            for k in range(5):
                mb[p][k][sl] = _lk(w2[25 + k] + h[0] * w2[k] + h[1] * w2[5 + k]
                                   + h[2] * w2[10 + k] + h[3] * w2[15 + k]
                                   + h[4] * w2[20 + k])
            return carry

        lax.fori_loop(0, GP, layer2, 0)

        w3 = [wbuf[55 + k, :] for k in range(30)]

        def layer3(g, carry):
            sl = pl.ds(g * 16, 16)
            h = [mb[p][k][sl] for k in range(5)]
            for k in range(5):
                mb[p][k][sl] = (w3[25 + k] + h[0] * w3[k] + h[1] * w3[5 + k]
                                + h[2] * w3[10 + k] + h[3] * w3[15 + k]
                                + h[4] * w3[20 + k])
            return carry

        lax.fori_loop(0, GP, layer3, 0)

    npairs = n_chunks // 2

    issue_idx(0, 0)
    wait_idx(0)
    issue_gathers(0)
    issue_idx(1, 1)

    def pair(i, carry):
        wait_gathers(0)
        wait_idx(1)
        issue_gathers(1)

        @pl.when(i > 0)
        def _():
            wait_scatters(0)
        compute(0)
        issue_scatters(0)

        @pl.when(i < npairs - 1)
        def _():
            issue_idx(2 * i + 2, 0)

        wait_gathers(1)

        @pl.when(i < npairs - 1)
        def _():
            wait_idx(0)
            issue_gathers(0)

        @pl.when(i > 0)
        def _():
            wait_scatters(1)
        compute(1)
        issue_scatters(1)

        @pl.when(i < npairs - 1)
        def _():
            issue_idx(2 * i + 3, 1)
        return carry

    lax.fori_loop(0, npairs, pair, 0)
    wait_scatters(0)
    wait_scatters(1)

    plsc.subcore_barrier()

    for k, ac in enumerate(acc + (ash, bsh)):
        for off, ln in segs:
            pltpu.sync_copy(ac.at[pl.ds(rs + off, ln)], stg.at[pl.ds(0, ln)])
            pltpu.sync_copy(stg.at[pl.ds(0, ln)],
                            out_hbm.at[pl.ds((c * 8 + k) * NR + rs + off, ln)])


def _fin_body(pref, ptref, oref):
    p3 = pref[...]
    pa = p3[0]
    pb = p3[1]
    p = pa[0:6] + pb[0:6]
    ptv = ptref[...]
    ax = pa[6:7]
    ay = pa[7:8]
    s0, s1, s2, s3, s4 = p[0:1], p[1:2], p[2:3], p[3:4], p[4:5]
    deg = p[5:6]
    scale = ptv / (ptv * deg)
    wsum = scale * deg
    valid = deg > np.float32(0.0)
    o0 = ptv * wsum
    ang = _TWO_PI * (scale * s4)
    cc = jnp.cos(ang)
    ss = jnp.sin(ang)
    za = ax * wsum
    zb = ay * wsum
    o5 = cc * za - ss * zb
    o6 = ss * za + cc * zb
    rows = jnp.concatenate(
        [o0, scale * s0, scale * s1, scale * s2, scale * s3, o5, o6,
         jnp.zeros_like(o0)], axis=0)
    oref[...] = jnp.where(valid, rows, np.float32(0.0))


def kernel(pt, features, angles, edge_index, W1, b1, W2, b2, W3, b3):
    N = pt.shape[0]
    E = edge_index.shape[1]

    assert E % (2 * NW * CH) == 0
    efl = edge_index.astype(jnp.int32).reshape(-1)
    EW = E // NW
    n_chunks = EW // CH

    ZR = -(-(N + 1) // (NS * 8)) * 8
    NR = NS * ZR
    fpl = jnp.pad(features[:, 0].astype(jnp.float32), (0, NR - N))
    apl = jnp.pad(angles[:, 0].astype(jnp.float32), (0, NR - N))
    bpl = jnp.pad(angles[:, 1].astype(jnp.float32), (0, NR - N))

    W1e = jnp.stack([W1[0] - W1[2], W1[1] + W1[2], W1[3], W1[4]])
    wvec = jnp.concatenate(
        [W1e.reshape(-1), b1, W2.reshape(-1), b2, W3.reshape(-1), b3,
         jnp.zeros((3,), jnp.float32)])
    wtab = jnp.tile(wvec[:, None], (1, 16))

    mesh = plsc.VectorSubcoreMesh(core_axis_name="c", subcore_axis_name="s")
    edge_fn = pl.kernel(
        functools.partial(_edge_body, NR=NR, EW=EW, n_chunks=n_chunks, ZR=ZR,
                          EOFF=E),
        out_type=jax.ShapeDtypeStruct((2 * 8 * NR,), jnp.float32),
        mesh=mesh,
        scratch_types=(
            [pltpu.VMEM((CH,), jnp.int32)] * 6
            + [pltpu.VMEM((CH,), jnp.float32)] * 12
            + [pltpu.VMEM((CH,), jnp.float32)] * 10
            + [pltpu.VMEM((CH,), jnp.float32)]
            + [pltpu.VMEM((88, 16), jnp.float32)]
            + [pltpu.VMEM_SHARED((NR,), jnp.float32)] * 9
            + [pltpu.SemaphoreType.DMA] * 26
        ),
    )
    partials = edge_fn(efl, fpl, apl, bpl, wtab)

    Np = NR
    P2 = partials.reshape(2, 8, NR)
    ptp = jnp.pad(pt.astype(jnp.float32), (0, Np - N)).reshape(1, Np)
    BN = Np // 2
    outt = pl.pallas_call(
        _fin_body,
        out_shape=jax.ShapeDtypeStruct((8, Np), jnp.float32),
        grid=(Np // BN,),
        in_specs=[
            pl.BlockSpec((2, 8, BN), lambda i: (0, 0, i)),
            pl.BlockSpec((1, BN), lambda i: (0, i)),
        ],
        out_specs=pl.BlockSpec((8, BN), lambda i: (0, i)),
    )(P2, ptp)
    return jnp.transpose(outt)[:N, :7]


# speedup vs baseline: 1.0054x; 1.0054x over previous
import functools

import numpy as np
import jax
import jax.numpy as jnp
from jax import lax
from jax.experimental import pallas as pl
from jax.experimental.pallas import tpu as pltpu
from jax.experimental.pallas import tpu_sc as plsc

NC = 2
NS = 16
NW = NC * NS
CH = 2000
GP = CH // 16

_TWO_PI = np.float32(2.0 * np.pi)


def _lk(v):
    return jnp.maximum(v, np.float32(0.01) * v)


def _edge_body(efl_hbm, f_hbm, a_hbm, b_hbm, w_hbm, out_hbm,
               *scr, NR, EW, n_chunks, ZR, EOFF):
    sidx = scr[0:2]
    didx = scr[2:4]
    dsc = scr[4:6]
    gb = (scr[6:12], scr[12:18])
    mb = (scr[18:23], scr[23:28])
    onesb, wbuf = scr[28:30]
    fsh, ash, bsh = scr[30:33]
    acc = scr[33:39]
    isem = scr[39:41]
    gsem = (scr[41:47], scr[47:53])
    ssem = (scr[53:59], scr[59:65])
    stg = gb[0][0]

    segs = []
    off = 0
    while off < ZR:
        segs.append((off, min(CH, ZR - off)))
        off += CH

    c = lax.axis_index("c")
    s = lax.axis_index("s")
    wid = c * NS + s

    rs = s * ZR
    for hsrc, shdst in ((f_hbm, fsh), (a_hbm, ash), (b_hbm, bsh)):
        for off, ln in segs:
            pltpu.sync_copy(hsrc.at[pl.ds(rs + off, ln)], stg.at[pl.ds(0, ln)])
            pltpu.sync_copy(stg.at[pl.ds(0, ln)], shdst.at[pl.ds(rs + off, ln)])

    def zloop(t, carry):
        stg[pl.ds(t * 16, 16)] = jnp.zeros((16,), jnp.float32)
        return carry
    lax.fori_loop(0, CH // 16, zloop, 0)
    for ac in acc:
        for off, ln in segs:
            pltpu.sync_copy(stg.at[pl.ds(0, ln)], ac.at[pl.ds(rs + off, ln)])

    pltpu.sync_copy(w_hbm, wbuf)
    for g in range(GP):
        onesb[pl.ds(g * 16, 16)] = jnp.ones((16,), jnp.float32)

    plsc.subcore_barrier()

    ebase = wid * EW

    def issue_idx(cid, p):
        base = ebase + cid * CH
        pltpu.async_copy(efl_hbm.at[pl.ds(base, CH)], sidx[p], isem[p])
        pltpu.async_copy(efl_hbm.at[pl.ds(EOFF + base, CH)], didx[p], isem[p])

    def wait_idx(p):
        pltpu.make_async_copy(efl_hbm.at[pl.ds(0, CH)], sidx[p], isem[p]).wait()
        pltpu.make_async_copy(efl_hbm.at[pl.ds(0, CH)], didx[p], isem[p]).wait()

    def gather_list(p):
        return ((fsh.at[didx[p]], gb[p][0]), (ash.at[didx[p]], gb[p][1]),
                (bsh.at[didx[p]], gb[p][2]), (fsh.at[sidx[p]], gb[p][3]),
                (ash.at[sidx[p]], gb[p][4]), (bsh.at[sidx[p]], gb[p][5]))

    def issue_gathers(p):
        for k, (s_, d_) in enumerate(gather_list(p)):
            pltpu.async_copy(s_, d_, gsem[p][k])

    def wait_gathers(p):
        for k, (s_, d_) in enumerate(gather_list(p)):
            pltpu.make_async_copy(s_, d_, gsem[p][k]).wait()

    def scatter_list(p):
        return ((mb[p][0], acc[0]), (mb[p][1], acc[1]), (mb[p][2], acc[2]),
                (mb[p][3], acc[3]), (mb[p][4], acc[4]), (onesb, acc[5]))

    def issue_scatters(p):
        for k, (s_, d_) in enumerate(scatter_list(p)):
            pltpu.async_copy(s_, d_.at[dsc[p]], ssem[p][k], add=True)

    def wait_scatters(p):
        for k, (s_, d_) in enumerate(scatter_list(p)):
            pltpu.make_async_copy(s_, d_.at[dsc[p]], ssem[p][k]).wait()

    def compute(p):
        gfi, gai, gbi, gfj, gaj, gbj = gb[p]

        w1 = [wbuf[k, :] for k in range(25)]

        def layer1(g, carry):
            sl = pl.ds(g * 16, 16)
            dsc[p][sl] = didx[p][sl]
            fi = gfi[sl]
            ai = gai[sl]
            bi = gbi[sl]
            fj = gfj[sl]
            aj = gaj[sl]
            bj = gbj[sl]
            inv = np.float32(1.0) / (aj * aj + bj * bj)
            re = (ai * aj + bi * bj) * inv
            im = (bi * aj - ai * bj) * inv
            for k in range(5):
                mb[p][k][sl] = _lk(w1[20 + k] + fi * w1[k] + fj * w1[5 + k]
                                   + re * w1[10 + k] + im * w1[15 + k])
            return carry

        lax.fori_loop(0, GP, layer1, 0)

        w2 = [wbuf[25 + k, :] for k in range(30)]

        def layer2(g, carry):
            sl = pl.ds(g * 16, 16)
            h = [mb[p][k][sl] for k in range(5)]
            for k in range(5):
                mb[p][k][sl] = _lk(w2[25 + k] + h[0] * w2[k] + h[1] * w2[5 + k]
                                   + h[2] * w2[10 + k] + h[3] * w2[15 + k]
                                   + h[4] * w2[20 + k])
            return carry

        lax.fori_loop(0, GP, layer2, 0)

        w3 = [wbuf[55 + k, :] for k in range(30)]

        def layer3(g, carry):
            sl = pl.ds(g * 16, 16)
            h = [mb[p][k][sl] for k in range(5)]
            for k in range(5):
                mb[p][k][sl] = (w3[25 + k] + h[0] * w3[k] + h[1] * w3[5 + k]
                                + h[2] * w3[10 + k] + h[3] * w3[15 + k]
                                + h[4] * w3[20 + k])
            return carry

        lax.fori_loop(0, GP, layer3, 0)

    npairs = n_chunks // 2

    issue_idx(0, 0)
    wait_idx(0)
    issue_gathers(0)
    issue_idx(1, 1)

    def pair(i, carry):
        wait_gathers(0)
        wait_idx(1)
        issue_gathers(1)

        @pl.when(i > 0)
        def _():
            wait_scatters(0)
        compute(0)
        issue_scatters(0)

        @pl.when(i < npairs - 1)
        def _():
            issue_idx(2 * i + 2, 0)

        wait_gathers(1)

        @pl.when(i < npairs - 1)
        def _():
            wait_idx(0)
            issue_gathers(0)

        @pl.when(i > 0)
        def _():
            wait_scatters(1)
        compute(1)
        issue_scatters(1)

        @pl.when(i < npairs - 1)
        def _():
            issue_idx(2 * i + 3, 1)
        return carry

    lax.fori_loop(0, npairs, pair, 0)
    wait_scatters(0)
    wait_scatters(1)

    plsc.subcore_barrier()

    for k, ac in enumerate(acc):
        for off, ln in segs:
            pltpu.sync_copy(ac.at[pl.ds(rs + off, ln)], stg.at[pl.ds(0, ln)])
            pltpu.sync_copy(stg.at[pl.ds(0, ln)],
                            out_hbm.at[pl.ds((c * 6 + k) * NR + rs + off, ln)])


def _fin_body(pref, ptref, aref, oref):
    p3 = pref[...]
    p = p3[0] + p3[1]
    ptv = ptref[...]
    a2 = aref[...]
    ax = a2[0:1]
    ay = a2[1:2]
    s0, s1, s2, s3, s4 = p[0:1], p[1:2], p[2:3], p[3:4], p[4:5]
    deg = p[5:6]
    scale = ptv / (ptv * deg)
    wsum = scale * deg
    valid = deg > np.float32(0.0)
    o0 = ptv * wsum
    ang = _TWO_PI * (scale * s4)
    cc = jnp.cos(ang)
    ss = jnp.sin(ang)
    za = ax * wsum
    zb = ay * wsum
    o5 = cc * za - ss * zb
    o6 = ss * za + cc * zb
    rows = jnp.concatenate(
        [o0, scale * s0, scale * s1, scale * s2, scale * s3, o5, o6,
         jnp.zeros_like(o0)], axis=0)
    oref[...] = jnp.where(valid, rows, np.float32(0.0))


def kernel(pt, features, angles, edge_index, W1, b1, W2, b2, W3, b3):
    N = pt.shape[0]
    E = edge_index.shape[1]

    assert E % (2 * NW * CH) == 0
    efl = edge_index.astype(jnp.int32).reshape(-1)
    EW = E // NW
    n_chunks = EW // CH

    ZR = -(-(N + 1) // (NS * 8)) * 8
    NR = NS * ZR
    fpl = jnp.pad(features[:, 0].astype(jnp.float32), (0, NR - N))
    apl = jnp.pad(angles[:, 0].astype(jnp.float32), (0, NR - N))
    bpl = jnp.pad(angles[:, 1].astype(jnp.float32), (0, NR - N))

    W1e = jnp.stack([W1[0] - W1[2], W1[1] + W1[2], W1[3], W1[4]])
    wvec = jnp.concatenate(
        [W1e.reshape(-1), b1, W2.reshape(-1), b2, W3.reshape(-1), b3,
         jnp.zeros((3,), jnp.float32)])
    wtab = jnp.tile(wvec[:, None], (1, 16))

    mesh = plsc.VectorSubcoreMesh(core_axis_name="c", subcore_axis_name="s")
    edge_fn = pl.kernel(
        functools.partial(_edge_body, NR=NR, EW=EW, n_chunks=n_chunks, ZR=ZR,
                          EOFF=E),
        out_type=jax.ShapeDtypeStruct((2 * 6 * NR,), jnp.float32),
        mesh=mesh,
        scratch_types=(
            [pltpu.VMEM((CH,), jnp.int32)] * 6
            + [pltpu.VMEM((CH,), jnp.float32)] * 12
            + [pltpu.VMEM((CH,), jnp.float32)] * 10
            + [pltpu.VMEM((CH,), jnp.float32)]
            + [pltpu.VMEM((88, 16), jnp.float32)]
            + [pltpu.VMEM_SHARED((NR,), jnp.float32)] * 9
            + [pltpu.SemaphoreType.DMA] * 26
        ),
    )
    partials = edge_fn(efl, fpl, apl, bpl, wtab)

    Np = NR
    P2 = partials.reshape(2, 6, NR)
    ptp = jnp.pad(pt.astype(jnp.float32), (0, Np - N)).reshape(1, Np)
    angp = jnp.pad(angles.astype(jnp.float32).T, ((0, 0), (0, Np - N)))
    BN = Np // 2
    outt = pl.pallas_call(
        _fin_body,
        out_shape=jax.ShapeDtypeStruct((8, Np), jnp.float32),
        grid=(Np // BN,),
        in_specs=[
            pl.BlockSpec((2, 6, BN), lambda i: (0, 0, i)),
            pl.BlockSpec((1, BN), lambda i: (0, i)),
            pl.BlockSpec((2, BN), lambda i: (0, i)),
        ],
        out_specs=pl.BlockSpec((8, BN), lambda i: (0, i)),
    )(P2, ptp, angp)
    return jnp.transpose(outt)[:N, :7]
